# baseline (speedup 1.0000x reference)
import functools
import math

import jax
import jax.numpy as jnp
from jax import lax
from jax.experimental import pallas as pl
from jax.experimental.pallas import tpu as pltpu
from jax.experimental.pallas import tpu_sc as plsc


def _pe_table(L, E):
  def body(o_ref):
    j = lax.broadcasted_iota(jnp.int32, (L, E), 1)
    pos = lax.broadcasted_iota(jnp.int32, (L, E), 0).astype(jnp.float32) + 1.0
    expnt = (2 * (j >> 1)).astype(jnp.float32) * (math.log(10000.0) / E)
    ang = pos * jnp.exp(-expnt)
    o_ref[...] = jnp.where(j % 2 == 0, jnp.cos(ang), jnp.sin(ang))

  return pl.pallas_call(
      body, out_shape=jax.ShapeDtypeStruct((L, E), jnp.float32))()


@functools.cache
def _make_emb(B, L, E):
  info = plsc.get_sparse_core_info()
  NC, NS = info.num_cores, info.num_subcores
  NW = NC * NS
  assert B % NW == 0
  seq_per_w = B // NW
  chunks = [(o, min(128, L - o)) for o in range(0, L, 128)]
  mesh = plsc.VectorSubcoreMesh(core_axis_name="c", subcore_axis_name="s")

  @functools.partial(
      pl.kernel,
      out_type=jax.ShapeDtypeStruct((B, L, E), jnp.float32),
      mesh=mesh,
      compiler_params=pltpu.CompilerParams(use_tc_tiling_on_sc=False),
      scratch_types=[
          pltpu.VMEM((L,), jnp.int32),
          pltpu.VMEM((L, E), jnp.float32),
          pltpu.VMEM((L, E), jnp.float32),
          pltpu.SemaphoreType.DMA,
      ],
  )
  def emb(x_hbm, w_hbm, pe_hbm, out_hbm, idx_v, rows_v, pe_v, sem):
    wid = lax.axis_index("s") * NC + lax.axis_index("c")
    seq0 = wid * seq_per_w
    pltpu.sync_copy(pe_hbm, pe_v)

    def seq_body(i, carry):
      seq = seq0 + i
      pltpu.sync_copy(x_hbm.at[pl.ds(seq * L, L)], idx_v)
      cps = [
          pltpu.async_copy(
              w_hbm.at[idx_v.at[pl.ds(o, n)]], rows_v.at[pl.ds(o, n)], sem)
          for (o, n) in chunks
      ]
      for cp in cps:
        cp.wait()

      def l_body(l, c):
        for jj in range(E // 16):
          sl = pl.ds(jj * 16, 16)
          plsc.addupdate(rows_v.at[l, sl], pe_v[l, sl])
        return c

      lax.fori_loop(0, L, l_body, 0)
      pltpu.sync_copy(rows_v, out_hbm.at[seq])
      return carry

    lax.fori_loop(0, seq_per_w, seq_body, 0)

  return emb


def kernel(x_batch, W):
  B, L = x_batch.shape
  _, E = W.shape
  pe = _pe_table(L, E)
  x = x_batch.astype(jnp.int32).reshape(B * L)
  return _make_emb(B, L, E)(x, W, pe)
